# baseline (speedup 1.0000x reference)
import math

import jax
import jax.numpy as jnp
import numpy as np
from jax.experimental import pallas as pl
from jax.experimental.pallas import tpu as pltpu

_B = 32
_C = 30
_FB = 4
_PLV = (_C * (_C - 1) // 2) * _FB
_TL = _PLV // _C
_NP = 32
_NG = 16
_BN_S = 1.0 / math.sqrt(1.0 + 1e-5)
_NEG = -1e30


def _pool_matrix(L, out_len):
    P = np.zeros((L, out_len), np.float32)
    for idx in range(out_len):
        s = (idx * L) // out_len
        e = ((idx + 1) * L + out_len - 1) // out_len
        P[s:e, idx] = 1.0 / (e - s)
    return P


_POOL1 = _pool_matrix(_TL, 100)
_POOL2 = _pool_matrix(100, 128)


def _elu(v):
    return jnp.where(v > 0, v, jnp.exp(v) - 1.0)


def _conv30(h, w, L):
    z = jnp.zeros((_C, 1), jnp.float32)
    hp = jnp.concatenate([z, h, z], axis=1)
    acc = jnp.dot(w[0], hp[:, 0:L], preferred_element_type=jnp.float32)
    acc += jnp.dot(w[1], hp[:, 1:L + 1], preferred_element_type=jnp.float32)
    acc += jnp.dot(w[2], hp[:, 2:L + 2], preferred_element_type=jnp.float32)
    return acc


def _edge_layer(nodes, w1, b1, w2, b2, bb, maskcol):
    d_in = nodes.shape[1]
    d = w2.shape[1]
    wt = w1[:d_in]
    wb = w1[d_in:]
    Bv = jnp.dot(nodes, wb, preferred_element_type=jnp.float32)
    A = jnp.dot(nodes, wt, preferred_element_type=jnp.float32) - Bv + b1
    Bv4 = Bv.reshape(_NG, _NP, 1, d)
    A4 = A.reshape(_NG, 1, _NP, d)
    P = jax.nn.relu(Bv4 + A4)
    M = jnp.dot(P.reshape(_NG * _NP * _NP, d), w2,
                preferred_element_type=jnp.float32) + b2
    M = jax.nn.relu(M) + maskcol
    M4 = M.reshape(_NG, _NP, _NP, d)[:, :_C]
    out = jnp.max(M4, axis=1) + bb
    pool = jnp.max(out[:, :_C], axis=1)
    return out.reshape(_NG * _NP, d), pool


def _main_kernel(h_ref, c1w_ref, g1_ref, bb1_ref, c2w_ref, g2_ref, bb2_ref,
                 p1_ref, p2_ref, mask_ref,
                 e1w1_ref, e1b1_ref, e1w2_ref, e1b2_ref, e1b_ref,
                 e2w1_ref, e2b1_ref, e2w2_ref, e2b2_ref, e2b_ref,
                 e3w1_ref, e3b1_ref, e3w2_ref, e3b2_ref, e3b_ref,
                 out_ref):
    zpad = jnp.zeros((_NP - _C, 128), jnp.float32)
    cols = []
    for q in range(_NG):
        h = h_ref[q]
        h = _conv30(h, c1w_ref[...], _TL)
        h = h * (g1_ref[...] * _BN_S) + bb1_ref[...]
        h = _elu(h)
        h = jnp.dot(h, p1_ref[...], preferred_element_type=jnp.float32)
        h = _conv30(h, c2w_ref[...], 100)
        h = h * (g2_ref[...] * _BN_S) + bb2_ref[...]
        h = _elu(h)
        h = jnp.dot(h, p2_ref[...], preferred_element_type=jnp.float32)
        cols.append(h)
        cols.append(zpad)
    nodes0 = jnp.concatenate(cols, axis=0)

    mask = mask_ref[...]
    x1, pl1 = _edge_layer(nodes0, e1w1_ref[...], e1b1_ref[...],
                          e1w2_ref[...], e1b2_ref[...], e1b_ref[...], mask)
    x2, pl2 = _edge_layer(x1, e2w1_ref[...], e2b1_ref[...],
                          e2w2_ref[...], e2b2_ref[...], e2b_ref[...], mask)
    _, pl3 = _edge_layer(x2, e3w1_ref[...], e3b1_ref[...],
                         e3w2_ref[...], e3b2_ref[...], e3b_ref[...], mask)
    out_ref[0] = jnp.concatenate([pl1, pl2, pl3], axis=1)


def _head_kernel(p_ref, w1_ref, b1_ref, w2_ref, b2_ref, out_ref):
    o = jnp.dot(p_ref[...], w1_ref[...], preferred_element_type=jnp.float32)
    o = jax.nn.relu(o + b1_ref[...])
    o = jnp.dot(o, w2_ref[...], preferred_element_type=jnp.float32)
    o = jax.nn.relu(o + b2_ref[...])
    out_ref[...] = o


def _full(shape):
    nd = len(shape)
    return pl.BlockSpec(shape, lambda g, _n=nd: (0,) * _n)


def kernel(x, b1_conv_w, b1_bn_g, b1_bn_b, b2_conv_w, b2_bn_g, b2_bn_b,
           c1_w1, c1_b1, c1_w2, c1_b2, c1_bn_g, c1_bn_b,
           c2_w1, c2_b1, c2_w2, c2_b2, c2_bn_g, c2_bn_b,
           c3_w1, c3_b1, c3_w2, c3_b2, c3_bn_g, c3_bn_b,
           l1_w, l1_b, l2_w, l2_b):
    bsz = x.shape[0]
    ti, tj = np.triu_indices(_C, k=1)
    feats = [x[:, i][:, ti, tj] for i in range(_FB)]
    h = jnp.concatenate(feats, axis=1).reshape(bsz, _C, _TL)

    c1w = jnp.transpose(b1_conv_w, (2, 0, 1))
    c2w = jnp.transpose(b2_conv_w, (2, 0, 1))
    g1 = b1_bn_g.reshape(_C, 1)
    bb1 = b1_bn_b.reshape(_C, 1)
    g2 = b2_bn_g.reshape(_C, 1)
    bb2 = b2_bn_b.reshape(_C, 1)

    row = lambda a: a.reshape(1, -1)
    s1 = c1_bn_g * _BN_S
    s2 = c2_bn_g * _BN_S
    s3 = c3_bn_g * _BN_S
    r = np.arange(_NG * _NP * _NP)
    maskcol = jnp.asarray(
        np.where(((r // _NP) % _NP) == (r % _NP), _NEG, 0.0)
        .astype(np.float32).reshape(-1, 1))
    operands = (
        h, c1w, g1, bb1, c2w, g2, bb2,
        jnp.asarray(_POOL1), jnp.asarray(_POOL2), maskcol,
        c1_w1, row(c1_b1), c1_w2 * s1[None, :], row(c1_b2 * s1),
        row(c1_bn_b),
        c2_w1, row(c2_b1), c2_w2 * s2[None, :], row(c2_b2 * s2),
        row(c2_bn_b),
        c3_w1, row(c3_b1), c3_w2 * s3[None, :], row(c3_b2 * s3),
        row(c3_bn_b),
    )
    in_specs = [pl.BlockSpec((_NG, _C, _TL), lambda g: (g, 0, 0))]
    in_specs += [_full(op.shape) for op in operands[1:]]

    ngrid = bsz // _NG
    pooled = pl.pallas_call(
        _main_kernel,
        grid=(ngrid,),
        in_specs=in_specs,
        out_specs=pl.BlockSpec((1, _NG, 896), lambda g: (g, 0, 0)),
        out_shape=jax.ShapeDtypeStruct((ngrid, _NG, 896), jnp.float32),
        compiler_params=pltpu.CompilerParams(
            dimension_semantics=("arbitrary",)),
    )(*operands)
    pooled = pooled.reshape(bsz, 896)

    out = pl.pallas_call(
        _head_kernel,
        grid=(1,),
        in_specs=[_full(pooled.shape), _full(l1_w.shape),
                  _full((1, 256)), _full(l2_w.shape), _full((1, 128))],
        out_specs=pl.BlockSpec((bsz, 128), lambda g: (0, 0)),
        out_shape=jax.ShapeDtypeStruct((bsz, 128), jnp.float32),
    )(pooled, l1_w, row(l1_b), l2_w, row(l2_b))
    return out


# speedup vs baseline: 1.0225x; 1.0225x over previous
import math

import jax
import jax.numpy as jnp
import numpy as np
from jax.experimental import pallas as pl
from jax.experimental.pallas import tpu as pltpu

_B = 32
_C = 30
_FB = 4
_PLV = (_C * (_C - 1) // 2) * _FB
_TL = _PLV // _C
_NP = 32
_NG = 8
_BN_S = 1.0 / math.sqrt(1.0 + 1e-5)
_NEG = -1e30


def _pool_matrix(L, out_len):
    P = np.zeros((L, out_len), np.float32)
    for idx in range(out_len):
        s = (idx * L) // out_len
        e = ((idx + 1) * L + out_len - 1) // out_len
        P[s:e, idx] = 1.0 / (e - s)
    return P


_POOL1 = _pool_matrix(_TL, 100)
_POOL2 = _pool_matrix(100, 128)


def _elu(v):
    return jnp.where(v > 0, v, jnp.exp(v) - 1.0)


def _conv30(h, w, L):
    z = jnp.zeros((_C, 1), jnp.float32)
    hp = jnp.concatenate([z, h, z], axis=1)
    acc = jnp.dot(w[0], hp[:, 0:L], preferred_element_type=jnp.float32)
    acc += jnp.dot(w[1], hp[:, 1:L + 1], preferred_element_type=jnp.float32)
    acc += jnp.dot(w[2], hp[:, 2:L + 2], preferred_element_type=jnp.float32)
    return acc


def _edge_layer(nodes, w1, b1, w2, b2, bb, maskcol):
    d_in = nodes.shape[1]
    d = w2.shape[1]
    wt = w1[:d_in]
    wb = w1[d_in:]
    Bv = jnp.dot(nodes, wb, preferred_element_type=jnp.float32)
    A = jnp.dot(nodes, wt, preferred_element_type=jnp.float32) - Bv + b1
    Bv4 = Bv.reshape(_NG, _NP, 1, d)
    A4 = A.reshape(_NG, 1, _NP, d)
    P = jax.nn.relu(Bv4 + A4).astype(jnp.bfloat16)
    M = jnp.dot(P.reshape(_NG * _NP * _NP, d), w2,
                preferred_element_type=jnp.float32) + b2
    M = jax.nn.relu(M) + maskcol
    M4 = M.reshape(_NG, _NP, _NP, d)[:, :_C]
    out = jnp.max(M4, axis=1) + bb
    pool = jnp.max(out[:, :_C], axis=1)
    return out.reshape(_NG * _NP, d), pool


def _main_kernel(h_ref, c1w_ref, g1_ref, bb1_ref, c2w_ref, g2_ref, bb2_ref,
                 p1_ref, p2_ref, mask_ref,
                 e1w1_ref, e1b1_ref, e1w2_ref, e1b2_ref, e1b_ref,
                 e2w1_ref, e2b1_ref, e2w2_ref, e2b2_ref, e2b_ref,
                 e3w1_ref, e3b1_ref, e3w2_ref, e3b2_ref, e3b_ref,
                 out_ref):
    zpad = jnp.zeros((_NP - _C, 128), jnp.float32)
    cols = []
    for q in range(_NG):
        h = h_ref[q]
        h = _conv30(h, c1w_ref[...], _TL)
        h = h * (g1_ref[...] * _BN_S) + bb1_ref[...]
        h = _elu(h)
        h = jnp.dot(h, p1_ref[...], preferred_element_type=jnp.float32)
        h = _conv30(h, c2w_ref[...], 100)
        h = h * (g2_ref[...] * _BN_S) + bb2_ref[...]
        h = _elu(h)
        h = jnp.dot(h, p2_ref[...], preferred_element_type=jnp.float32)
        cols.append(h)
        cols.append(zpad)
    nodes0 = jnp.concatenate(cols, axis=0)

    mask = mask_ref[...]
    x1, pl1 = _edge_layer(nodes0, e1w1_ref[...], e1b1_ref[...],
                          e1w2_ref[...], e1b2_ref[...], e1b_ref[...], mask)
    x2, pl2 = _edge_layer(x1, e2w1_ref[...], e2b1_ref[...],
                          e2w2_ref[...], e2b2_ref[...], e2b_ref[...], mask)
    _, pl3 = _edge_layer(x2, e3w1_ref[...], e3b1_ref[...],
                         e3w2_ref[...], e3b2_ref[...], e3b_ref[...], mask)
    out_ref[0] = jnp.concatenate([pl1, pl2, pl3], axis=1)


def _head_kernel(p_ref, w1_ref, b1_ref, w2_ref, b2_ref, out_ref):
    o = jnp.dot(p_ref[...], w1_ref[...], preferred_element_type=jnp.float32)
    o = jax.nn.relu(o + b1_ref[...])
    o = jnp.dot(o, w2_ref[...], preferred_element_type=jnp.float32)
    o = jax.nn.relu(o + b2_ref[...])
    out_ref[...] = o


def _full(shape):
    nd = len(shape)
    return pl.BlockSpec(shape, lambda g, _n=nd: (0,) * _n)


def kernel(x, b1_conv_w, b1_bn_g, b1_bn_b, b2_conv_w, b2_bn_g, b2_bn_b,
           c1_w1, c1_b1, c1_w2, c1_b2, c1_bn_g, c1_bn_b,
           c2_w1, c2_b1, c2_w2, c2_b2, c2_bn_g, c2_bn_b,
           c3_w1, c3_b1, c3_w2, c3_b2, c3_bn_g, c3_bn_b,
           l1_w, l1_b, l2_w, l2_b):
    bsz = x.shape[0]
    ti, tj = np.triu_indices(_C, k=1)
    feats = [x[:, i][:, ti, tj] for i in range(_FB)]
    h = jnp.concatenate(feats, axis=1).reshape(bsz, _C, _TL)

    c1w = jnp.transpose(b1_conv_w, (2, 0, 1))
    c2w = jnp.transpose(b2_conv_w, (2, 0, 1))
    g1 = b1_bn_g.reshape(_C, 1)
    bb1 = b1_bn_b.reshape(_C, 1)
    g2 = b2_bn_g.reshape(_C, 1)
    bb2 = b2_bn_b.reshape(_C, 1)

    row = lambda a: a.reshape(1, -1)
    s1 = c1_bn_g * _BN_S
    s2 = c2_bn_g * _BN_S
    s3 = c3_bn_g * _BN_S
    r = np.arange(_NG * _NP * _NP)
    maskcol = jnp.asarray(
        np.where(((r // _NP) % _NP) == (r % _NP), _NEG, 0.0)
        .astype(np.float32).reshape(-1, 1))
    operands = (
        h, c1w, g1, bb1, c2w, g2, bb2,
        jnp.asarray(_POOL1), jnp.asarray(_POOL2), maskcol,
        c1_w1, row(c1_b1),
        (c1_w2 * s1[None, :]).astype(jnp.bfloat16), row(c1_b2 * s1),
        row(c1_bn_b),
        c2_w1, row(c2_b1),
        (c2_w2 * s2[None, :]).astype(jnp.bfloat16), row(c2_b2 * s2),
        row(c2_bn_b),
        c3_w1, row(c3_b1),
        (c3_w2 * s3[None, :]).astype(jnp.bfloat16), row(c3_b2 * s3),
        row(c3_bn_b),
    )
    in_specs = [pl.BlockSpec((_NG, _C, _TL), lambda g: (g, 0, 0))]
    in_specs += [_full(op.shape) for op in operands[1:]]

    ngrid = bsz // _NG
    pooled = pl.pallas_call(
        _main_kernel,
        grid=(ngrid,),
        in_specs=in_specs,
        out_specs=pl.BlockSpec((1, _NG, 896), lambda g: (g, 0, 0)),
        out_shape=jax.ShapeDtypeStruct((ngrid, _NG, 896), jnp.float32),
        compiler_params=pltpu.CompilerParams(
            dimension_semantics=("arbitrary",)),
    )(*operands)
    pooled = pooled.reshape(bsz, 896)

    out = pl.pallas_call(
        _head_kernel,
        grid=(1,),
        in_specs=[_full(pooled.shape), _full(l1_w.shape),
                  _full((1, 256)), _full(l2_w.shape), _full((1, 128))],
        out_specs=pl.BlockSpec((bsz, 128), lambda g: (0, 0)),
        out_shape=jax.ShapeDtypeStruct((bsz, 128), jnp.float32),
    )(pooled, l1_w, row(l1_b), l2_w, row(l2_b))
    return out


# speedup vs baseline: 1.0668x; 1.0433x over previous
import math

import jax
import jax.numpy as jnp
import numpy as np
from jax.experimental import pallas as pl
from jax.experimental.pallas import tpu as pltpu

_B = 32
_C = 30
_FB = 4
_PLV = (_C * (_C - 1) // 2) * _FB
_TL = _PLV // _C
_NP = 32
_NG = 8
_BN_S = 1.0 / math.sqrt(1.0 + 1e-5)
_NEG = -1e30


def _pool_matrix(L, out_len):
    P = np.zeros((L, out_len), np.float32)
    for idx in range(out_len):
        s = (idx * L) // out_len
        e = ((idx + 1) * L + out_len - 1) // out_len
        P[s:e, idx] = 1.0 / (e - s)
    return P


_POOL1 = _pool_matrix(_TL, 100)
_POOL2 = _pool_matrix(100, 128)


def _elu(v):
    return jnp.where(v > 0, v, jnp.exp(v) - 1.0)


def _conv30(h, w, L):
    z = jnp.zeros((_C, 1), jnp.float32)
    hp = jnp.concatenate([z, h, z], axis=1)
    acc = jnp.dot(w[0], hp[:, 0:L], preferred_element_type=jnp.float32)
    acc += jnp.dot(w[1], hp[:, 1:L + 1], preferred_element_type=jnp.float32)
    acc += jnp.dot(w[2], hp[:, 2:L + 2], preferred_element_type=jnp.float32)
    return acc


def _edge_layer(nodes, w1, b1, w2, combo, bb):
    d_in = nodes.shape[1]
    d = w2.shape[1]
    wt = w1[:d_in]
    wb = w1[d_in:]
    Bv = jnp.dot(nodes, wb, preferred_element_type=jnp.float32)
    A = jnp.dot(nodes, wt, preferred_element_type=jnp.float32) - Bv + b1
    Bv4 = Bv.reshape(_NG, _NP, 1, d)[:, :_C]
    A4 = A.reshape(_NG, 1, _NP, d)
    P = jax.nn.relu(Bv4 + A4).astype(jnp.bfloat16)
    M = jnp.dot(P.reshape(_NG * _C * _NP, d), w2,
                preferred_element_type=jnp.float32)
    M4 = M.reshape(_NG, _C, _NP, d) + combo[None]
    out = jax.nn.relu(jnp.max(M4, axis=1)) + bb
    pool = jnp.max(out[:, :_C], axis=1)
    return out.reshape(_NG * _NP, d), pool


def _main_kernel(h_ref, c1w_ref, g1_ref, bb1_ref, c2w_ref, g2_ref, bb2_ref,
                 p1_ref, p2_ref,
                 e1w1_ref, e1b1_ref, e1w2_ref, e1c_ref, e1b_ref,
                 e2w1_ref, e2b1_ref, e2w2_ref, e2c_ref, e2b_ref,
                 e3w1_ref, e3b1_ref, e3w2_ref, e3c_ref, e3b_ref,
                 out_ref):
    zpad = jnp.zeros((_NP - _C, 128), jnp.float32)
    cols = []
    for q in range(_NG):
        h = h_ref[q]
        h = _conv30(h, c1w_ref[...], _TL)
        h = h * (g1_ref[...] * _BN_S) + bb1_ref[...]
        h = _elu(h)
        h = jnp.dot(h, p1_ref[...], preferred_element_type=jnp.float32)
        h = _conv30(h, c2w_ref[...], 100)
        h = h * (g2_ref[...] * _BN_S) + bb2_ref[...]
        h = _elu(h)
        h = jnp.dot(h, p2_ref[...], preferred_element_type=jnp.float32)
        cols.append(h)
        cols.append(zpad)
    nodes0 = jnp.concatenate(cols, axis=0)

    x1, pl1 = _edge_layer(nodes0, e1w1_ref[...], e1b1_ref[...],
                          e1w2_ref[...], e1c_ref[...], e1b_ref[...])
    x2, pl2 = _edge_layer(x1, e2w1_ref[...], e2b1_ref[...],
                          e2w2_ref[...], e2c_ref[...], e2b_ref[...])
    _, pl3 = _edge_layer(x2, e3w1_ref[...], e3b1_ref[...],
                         e3w2_ref[...], e3c_ref[...], e3b_ref[...])
    out_ref[0] = jnp.concatenate([pl1, pl2, pl3], axis=1)


def _head_kernel(p_ref, w1_ref, b1_ref, w2_ref, b2_ref, out_ref):
    o = jnp.dot(p_ref[...], w1_ref[...], preferred_element_type=jnp.float32)
    o = jax.nn.relu(o + b1_ref[...])
    o = jnp.dot(o, w2_ref[...], preferred_element_type=jnp.float32)
    o = jax.nn.relu(o + b2_ref[...])
    out_ref[...] = o


def _full(shape):
    nd = len(shape)
    return pl.BlockSpec(shape, lambda g, _n=nd: (0,) * _n)


def kernel(x, b1_conv_w, b1_bn_g, b1_bn_b, b2_conv_w, b2_bn_g, b2_bn_b,
           c1_w1, c1_b1, c1_w2, c1_b2, c1_bn_g, c1_bn_b,
           c2_w1, c2_b1, c2_w2, c2_b2, c2_bn_g, c2_bn_b,
           c3_w1, c3_b1, c3_w2, c3_b2, c3_bn_g, c3_bn_b,
           l1_w, l1_b, l2_w, l2_b):
    bsz = x.shape[0]
    ti, tj = np.triu_indices(_C, k=1)
    feats = [x[:, i][:, ti, tj] for i in range(_FB)]
    h = jnp.concatenate(feats, axis=1).reshape(bsz, _C, _TL)

    c1w = jnp.transpose(b1_conv_w, (2, 0, 1))
    c2w = jnp.transpose(b2_conv_w, (2, 0, 1))
    g1 = b1_bn_g.reshape(_C, 1)
    bb1 = b1_bn_b.reshape(_C, 1)
    g2 = b2_bn_g.reshape(_C, 1)
    bb2 = b2_bn_b.reshape(_C, 1)

    row = lambda a: a.reshape(1, -1)
    s1 = c1_bn_g * _BN_S
    s2 = c2_bn_g * _BN_S
    s3 = c3_bn_g * _BN_S
    diag = np.where(np.arange(_C)[:, None] == np.arange(_NP)[None, :],
                    np.float32(_NEG), np.float32(0.0))
    combo = lambda b2s: b2s[None, None, :] + diag[:, :, None]
    operands = (
        h, c1w, g1, bb1, c2w, g2, bb2,
        jnp.asarray(_POOL1), jnp.asarray(_POOL2),
        c1_w1, row(c1_b1),
        (c1_w2 * s1[None, :]).astype(jnp.bfloat16), combo(c1_b2 * s1),
        row(c1_bn_b),
        c2_w1, row(c2_b1),
        (c2_w2 * s2[None, :]).astype(jnp.bfloat16), combo(c2_b2 * s2),
        row(c2_bn_b),
        c3_w1, row(c3_b1),
        (c3_w2 * s3[None, :]).astype(jnp.bfloat16), combo(c3_b2 * s3),
        row(c3_bn_b),
    )
    in_specs = [pl.BlockSpec((_NG, _C, _TL), lambda g: (g, 0, 0))]
    in_specs += [_full(op.shape) for op in operands[1:]]

    ngrid = bsz // _NG
    pooled = pl.pallas_call(
        _main_kernel,
        grid=(ngrid,),
        in_specs=in_specs,
        out_specs=pl.BlockSpec((1, _NG, 896), lambda g: (g, 0, 0)),
        out_shape=jax.ShapeDtypeStruct((ngrid, _NG, 896), jnp.float32),
        compiler_params=pltpu.CompilerParams(
            dimension_semantics=("arbitrary",)),
    )(*operands)
    pooled = pooled.reshape(bsz, 896)

    out = pl.pallas_call(
        _head_kernel,
        grid=(1,),
        in_specs=[_full(pooled.shape), _full(l1_w.shape),
                  _full((1, 256)), _full(l2_w.shape), _full((1, 128))],
        out_specs=pl.BlockSpec((bsz, 128), lambda g: (0, 0)),
        out_shape=jax.ShapeDtypeStruct((bsz, 128), jnp.float32),
    )(pooled, l1_w, row(l1_b), l2_w, row(l2_b))
    return out


# speedup vs baseline: 1.4035x; 1.3156x over previous
import math

import jax
import jax.numpy as jnp
import numpy as np
from jax.experimental import pallas as pl
from jax.experimental.pallas import tpu as pltpu

_B = 32
_C = 30
_FB = 4
_PLV = (_C * (_C - 1) // 2) * _FB
_TL = _PLV // _C
_NP = 32
_NG = 8
_BN_S = 1.0 / math.sqrt(1.0 + 1e-5)
_NEG = -1e30


def _pool_matrix(L, out_len):
    P = np.zeros((L, out_len), np.float32)
    for idx in range(out_len):
        s = (idx * L) // out_len
        e = ((idx + 1) * L + out_len - 1) // out_len
        P[s:e, idx] = 1.0 / (e - s)
    return P


_POOL1 = _pool_matrix(_TL, 100)
_POOL2 = _pool_matrix(100, 128)


def _elu(v):
    return jnp.where(v > 0, v, jnp.exp(v) - 1.0)


def _conv30(h, w, L):
    z = jnp.zeros((_C, 1), jnp.float32)
    hp = jnp.concatenate([z, h, z], axis=1)
    acc = jnp.dot(w[0], hp[:, 0:L], preferred_element_type=jnp.float32)
    acc += jnp.dot(w[1], hp[:, 1:L + 1], preferred_element_type=jnp.float32)
    acc += jnp.dot(w[2], hp[:, 2:L + 2], preferred_element_type=jnp.float32)
    return acc


def _edge_layer(nodes, w1, b1, w2, combo, bb):
    d_in = nodes.shape[1]
    d = w2.shape[1]
    wt = w1[:d_in]
    wb = w1[d_in:]
    Bv = jnp.dot(nodes, wb, preferred_element_type=jnp.float32)
    A = jnp.dot(nodes, wt, preferred_element_type=jnp.float32) - Bv + b1
    Bv4 = Bv.reshape(_NG, _NP, 1, d)[:, :_C]
    A4 = A.reshape(_NG, 1, _NP, d)
    P = jax.nn.relu(Bv4 + A4).astype(jnp.bfloat16)
    M = jnp.dot(P.reshape(_NG * _C * _NP, d), w2,
                preferred_element_type=jnp.float32)
    M4 = M.reshape(_NG, _C, _NP, d) + combo[None]
    out = jax.nn.relu(jnp.max(M4, axis=1)) + bb
    pool = jnp.max(out[:, :_C], axis=1)
    return out.reshape(_NG * _NP, d), pool


def _main_kernel(h_ref, c1w_ref, g1_ref, bb1_ref, c2w_ref, g2_ref, bb2_ref,
                 p1_ref, p2_ref,
                 e1w1_ref, e1b1_ref, e1w2_ref, e1c_ref, e1b_ref,
                 e2w1_ref, e2b1_ref, e2w2_ref, e2c_ref, e2b_ref,
                 e3w1_ref, e3b1_ref, e3w2_ref, e3c_ref, e3b_ref,
                 out_ref):
    zpad = jnp.zeros((_NP - _C, 128), jnp.float32)
    cols = []
    for q in range(_NG):
        h = h_ref[q]
        h = _conv30(h, c1w_ref[...], _TL)
        h = h * (g1_ref[...] * _BN_S) + bb1_ref[...]
        h = _elu(h)
        h = jnp.dot(h, p1_ref[...], preferred_element_type=jnp.float32)
        h = _conv30(h, c2w_ref[...], 100)
        h = h * (g2_ref[...] * _BN_S) + bb2_ref[...]
        h = _elu(h)
        h = jnp.dot(h, p2_ref[...], preferred_element_type=jnp.float32)
        cols.append(h)
        cols.append(zpad)
    nodes0 = jnp.concatenate(cols, axis=0)

    x1, pl1 = _edge_layer(nodes0, e1w1_ref[...], e1b1_ref[...],
                          e1w2_ref[...], e1c_ref[...], e1b_ref[...])
    x2, pl2 = _edge_layer(x1, e2w1_ref[...], e2b1_ref[...],
                          e2w2_ref[...], e2c_ref[...], e2b_ref[...])
    _, pl3 = _edge_layer(x2, e3w1_ref[...], e3b1_ref[...],
                         e3w2_ref[...], e3c_ref[...], e3b_ref[...])
    out_ref[0] = jnp.concatenate([pl1, pl2, pl3], axis=1)


def _head_kernel(p_ref, w1_ref, b1_ref, w2_ref, b2_ref, out_ref):
    o = jnp.dot(p_ref[...], w1_ref[...], preferred_element_type=jnp.float32)
    o = jax.nn.relu(o + b1_ref[...])
    o = jnp.dot(o, w2_ref[...], preferred_element_type=jnp.float32)
    o = jax.nn.relu(o + b2_ref[...])
    out_ref[...] = o


def _full(shape):
    nd = len(shape)
    return pl.BlockSpec(shape, lambda g, _n=nd: (0,) * _n)


def kernel(x, b1_conv_w, b1_bn_g, b1_bn_b, b2_conv_w, b2_bn_g, b2_bn_b,
           c1_w1, c1_b1, c1_w2, c1_b2, c1_bn_g, c1_bn_b,
           c2_w1, c2_b1, c2_w2, c2_b2, c2_bn_g, c2_bn_b,
           c3_w1, c3_b1, c3_w2, c3_b2, c3_bn_g, c3_bn_b,
           l1_w, l1_b, l2_w, l2_b):
    bsz = x.shape[0]
    ti, tj = np.triu_indices(_C, k=1)
    flat_idx = (np.arange(_FB)[:, None] * (_C * _C)
                + (ti * _C + tj)[None, :]).reshape(-1)
    h = x.reshape(bsz, _FB * _C * _C)[:, flat_idx].reshape(bsz, _C, _TL)

    c1w = jnp.transpose(b1_conv_w, (2, 0, 1))
    c2w = jnp.transpose(b2_conv_w, (2, 0, 1))
    g1 = b1_bn_g.reshape(_C, 1)
    bb1 = b1_bn_b.reshape(_C, 1)
    g2 = b2_bn_g.reshape(_C, 1)
    bb2 = b2_bn_b.reshape(_C, 1)

    row = lambda a: a.reshape(1, -1)
    s1 = c1_bn_g * _BN_S
    s2 = c2_bn_g * _BN_S
    s3 = c3_bn_g * _BN_S
    diag = np.where(np.arange(_C)[:, None] == np.arange(_NP)[None, :],
                    np.float32(_NEG), np.float32(0.0))
    combo = lambda b2s: b2s[None, None, :] + diag[:, :, None]
    operands = (
        h, c1w, g1, bb1, c2w, g2, bb2,
        jnp.asarray(_POOL1), jnp.asarray(_POOL2),
        c1_w1, row(c1_b1),
        (c1_w2 * s1[None, :]).astype(jnp.bfloat16), combo(c1_b2 * s1),
        row(c1_bn_b),
        c2_w1, row(c2_b1),
        (c2_w2 * s2[None, :]).astype(jnp.bfloat16), combo(c2_b2 * s2),
        row(c2_bn_b),
        c3_w1, row(c3_b1),
        (c3_w2 * s3[None, :]).astype(jnp.bfloat16), combo(c3_b2 * s3),
        row(c3_bn_b),
    )
    in_specs = [pl.BlockSpec((_NG, _C, _TL), lambda g: (g, 0, 0))]
    in_specs += [_full(op.shape) for op in operands[1:]]

    ngrid = bsz // _NG
    pooled = pl.pallas_call(
        _main_kernel,
        grid=(ngrid,),
        in_specs=in_specs,
        out_specs=pl.BlockSpec((1, _NG, 896), lambda g: (g, 0, 0)),
        out_shape=jax.ShapeDtypeStruct((ngrid, _NG, 896), jnp.float32),
        compiler_params=pltpu.CompilerParams(
            dimension_semantics=("arbitrary",)),
    )(*operands)
    pooled = pooled.reshape(bsz, 896)

    out = pl.pallas_call(
        _head_kernel,
        grid=(1,),
        in_specs=[_full(pooled.shape), _full(l1_w.shape),
                  _full((1, 256)), _full(l2_w.shape), _full((1, 128))],
        out_specs=pl.BlockSpec((bsz, 128), lambda g: (0, 0)),
        out_shape=jax.ShapeDtypeStruct((bsz, 128), jnp.float32),
    )(pooled, l1_w, row(l1_b), l2_w, row(l2_b))
    return out


# speedup vs baseline: 1.4476x; 1.0314x over previous
import math

import jax
import jax.numpy as jnp
import numpy as np
from jax.experimental import pallas as pl
from jax.experimental.pallas import tpu as pltpu

_B = 32
_C = 30
_FB = 4
_PLV = (_C * (_C - 1) // 2) * _FB
_TL = _PLV // _C
_NP = 32
_NG = 16
_BN_S = 1.0 / math.sqrt(1.0 + 1e-5)
_NEG = -1e30


def _pool_matrix(L, out_len):
    P = np.zeros((L, out_len), np.float32)
    for idx in range(out_len):
        s = (idx * L) // out_len
        e = ((idx + 1) * L + out_len - 1) // out_len
        P[s:e, idx] = 1.0 / (e - s)
    return P


_POOL1 = _pool_matrix(_TL, 100)
_POOL2 = _pool_matrix(100, 128)


def _elu(v):
    return jnp.where(v > 0, v, jnp.exp(v) - 1.0)


def _conv30(h, w, L):
    z = jnp.zeros((_C, 1), jnp.float32)
    hp = jnp.concatenate([z, h, z], axis=1)
    acc = jnp.dot(w[0], hp[:, 0:L], preferred_element_type=jnp.float32)
    acc += jnp.dot(w[1], hp[:, 1:L + 1], preferred_element_type=jnp.float32)
    acc += jnp.dot(w[2], hp[:, 2:L + 2], preferred_element_type=jnp.float32)
    return acc


def _edge_layer(nodes, w1, b1, w2, combo, bb):
    d_in = nodes.shape[1]
    d = w2.shape[1]
    wt = w1[:d_in]
    wb = w1[d_in:]
    Bv = jnp.dot(nodes, wb, preferred_element_type=jnp.float32)
    A = jnp.dot(nodes, wt, preferred_element_type=jnp.float32) - Bv + b1
    Bv4 = Bv.reshape(_NG, _NP, 1, d)[:, :_C]
    A4 = A.reshape(_NG, 1, _NP, d)
    P = jax.nn.relu(Bv4 + A4).astype(jnp.bfloat16)
    M = jnp.dot(P.reshape(_NG * _C * _NP, d), w2,
                preferred_element_type=jnp.float32)
    M4 = M.reshape(_NG, _C, _NP, d) + combo[None]
    out = jax.nn.relu(jnp.max(M4, axis=1)) + bb
    pool = jnp.max(out[:, :_C], axis=1)
    return out.reshape(_NG * _NP, d), pool


def _main_kernel(h_ref, c1w_ref, g1_ref, bb1_ref, c2w_ref, g2_ref, bb2_ref,
                 p1_ref, p2_ref,
                 e1w1_ref, e1b1_ref, e1w2_ref, e1c_ref, e1b_ref,
                 e2w1_ref, e2b1_ref, e2w2_ref, e2c_ref, e2b_ref,
                 e3w1_ref, e3b1_ref, e3w2_ref, e3c_ref, e3b_ref,
                 out_ref):
    zpad = jnp.zeros((_NP - _C, 128), jnp.float32)
    cols = []
    for q in range(_NG):
        h = h_ref[q]
        h = _conv30(h, c1w_ref[...], _TL)
        h = h * (g1_ref[...] * _BN_S) + bb1_ref[...]
        h = _elu(h)
        h = jnp.dot(h, p1_ref[...], preferred_element_type=jnp.float32)
        h = _conv30(h, c2w_ref[...], 100)
        h = h * (g2_ref[...] * _BN_S) + bb2_ref[...]
        h = _elu(h)
        h = jnp.dot(h, p2_ref[...], preferred_element_type=jnp.float32)
        cols.append(h)
        cols.append(zpad)
    nodes0 = jnp.concatenate(cols, axis=0)

    x1, pl1 = _edge_layer(nodes0, e1w1_ref[...], e1b1_ref[...],
                          e1w2_ref[...], e1c_ref[...], e1b_ref[...])
    x2, pl2 = _edge_layer(x1, e2w1_ref[...], e2b1_ref[...],
                          e2w2_ref[...], e2c_ref[...], e2b_ref[...])
    _, pl3 = _edge_layer(x2, e3w1_ref[...], e3b1_ref[...],
                         e3w2_ref[...], e3c_ref[...], e3b_ref[...])
    out_ref[0] = jnp.concatenate([pl1, pl2, pl3], axis=1)


def _head_kernel(p_ref, w1_ref, b1_ref, w2_ref, b2_ref, out_ref):
    o = jnp.dot(p_ref[...], w1_ref[...], preferred_element_type=jnp.float32)
    o = jax.nn.relu(o + b1_ref[...])
    o = jnp.dot(o, w2_ref[...], preferred_element_type=jnp.float32)
    o = jax.nn.relu(o + b2_ref[...])
    out_ref[...] = o


def _full(shape):
    nd = len(shape)
    return pl.BlockSpec(shape, lambda g, _n=nd: (0,) * _n)


def kernel(x, b1_conv_w, b1_bn_g, b1_bn_b, b2_conv_w, b2_bn_g, b2_bn_b,
           c1_w1, c1_b1, c1_w2, c1_b2, c1_bn_g, c1_bn_b,
           c2_w1, c2_b1, c2_w2, c2_b2, c2_bn_g, c2_bn_b,
           c3_w1, c3_b1, c3_w2, c3_b2, c3_bn_g, c3_bn_b,
           l1_w, l1_b, l2_w, l2_b):
    bsz = x.shape[0]
    ti, tj = np.triu_indices(_C, k=1)
    flat_idx = (np.arange(_FB)[:, None] * (_C * _C)
                + (ti * _C + tj)[None, :]).reshape(-1)
    h = x.reshape(bsz, _FB * _C * _C)[:, flat_idx].reshape(bsz, _C, _TL)

    c1w = jnp.transpose(b1_conv_w, (2, 0, 1))
    c2w = jnp.transpose(b2_conv_w, (2, 0, 1))
    g1 = b1_bn_g.reshape(_C, 1)
    bb1 = b1_bn_b.reshape(_C, 1)
    g2 = b2_bn_g.reshape(_C, 1)
    bb2 = b2_bn_b.reshape(_C, 1)

    row = lambda a: a.reshape(1, -1)
    s1 = c1_bn_g * _BN_S
    s2 = c2_bn_g * _BN_S
    s3 = c3_bn_g * _BN_S
    diag = np.where(np.arange(_C)[:, None] == np.arange(_NP)[None, :],
                    np.float32(_NEG), np.float32(0.0))
    combo = lambda b2s: b2s[None, None, :] + diag[:, :, None]
    operands = (
        h, c1w, g1, bb1, c2w, g2, bb2,
        jnp.asarray(_POOL1), jnp.asarray(_POOL2),
        c1_w1, row(c1_b1),
        (c1_w2 * s1[None, :]).astype(jnp.bfloat16), combo(c1_b2 * s1),
        row(c1_bn_b),
        c2_w1, row(c2_b1),
        (c2_w2 * s2[None, :]).astype(jnp.bfloat16), combo(c2_b2 * s2),
        row(c2_bn_b),
        c3_w1, row(c3_b1),
        (c3_w2 * s3[None, :]).astype(jnp.bfloat16), combo(c3_b2 * s3),
        row(c3_bn_b),
    )
    in_specs = [pl.BlockSpec((_NG, _C, _TL), lambda g: (g, 0, 0))]
    in_specs += [_full(op.shape) for op in operands[1:]]

    ngrid = bsz // _NG
    pooled = pl.pallas_call(
        _main_kernel,
        grid=(ngrid,),
        in_specs=in_specs,
        out_specs=pl.BlockSpec((1, _NG, 896), lambda g: (g, 0, 0)),
        out_shape=jax.ShapeDtypeStruct((ngrid, _NG, 896), jnp.float32),
        compiler_params=pltpu.CompilerParams(
            dimension_semantics=("arbitrary",)),
    )(*operands)
    pooled = pooled.reshape(bsz, 896)

    out = pl.pallas_call(
        _head_kernel,
        grid=(1,),
        in_specs=[_full(pooled.shape), _full(l1_w.shape),
                  _full((1, 256)), _full(l2_w.shape), _full((1, 128))],
        out_specs=pl.BlockSpec((bsz, 128), lambda g: (0, 0)),
        out_shape=jax.ShapeDtypeStruct((bsz, 128), jnp.float32),
    )(pooled, l1_w, row(l1_b), l2_w, row(l2_b))
    return out


# speedup vs baseline: 1.4877x; 1.0277x over previous
import math

import jax
import jax.numpy as jnp
import numpy as np
from jax.experimental import pallas as pl
from jax.experimental.pallas import tpu as pltpu

_B = 32
_C = 30
_FB = 4
_PLV = (_C * (_C - 1) // 2) * _FB
_TL = _PLV // _C
_NP = 32
_NG = 16
_BN_S = 1.0 / math.sqrt(1.0 + 1e-5)
_NEG = -1e30


def _pool_matrix(L, out_len):
    P = np.zeros((L, out_len), np.float32)
    for idx in range(out_len):
        s = (idx * L) // out_len
        e = ((idx + 1) * L + out_len - 1) // out_len
        P[s:e, idx] = 1.0 / (e - s)
    return P


_POOL1 = _pool_matrix(_TL, 100)
_POOL2 = _pool_matrix(100, 128)


def _elu(v):
    return jnp.where(v > 0, v, jnp.exp(v) - 1.0)


def _conv30(h, w, L):
    z = jnp.zeros((_C, 1), jnp.float32)
    hp = jnp.concatenate([z, h, z], axis=1)
    acc = jnp.dot(w[0], hp[:, 0:L], preferred_element_type=jnp.float32)
    acc += jnp.dot(w[1], hp[:, 1:L + 1], preferred_element_type=jnp.float32)
    acc += jnp.dot(w[2], hp[:, 2:L + 2], preferred_element_type=jnp.float32)
    return acc


def _edge_layer(nodes, w1, b1, w2, combo, bb):
    d_in = nodes.shape[1]
    d = w2.shape[1]
    wt = w1[:d_in]
    wb = w1[d_in:]
    Bv = jnp.dot(nodes, wb, preferred_element_type=jnp.float32)
    A = jnp.dot(nodes, wt, preferred_element_type=jnp.float32) - Bv + b1
    Bv4 = Bv.reshape(_NG, _NP, 1, d)[:, :_C]
    A4 = A.reshape(_NG, 1, _NP, d)
    P = jax.nn.relu(Bv4 + A4).astype(jnp.bfloat16)
    M = jnp.dot(P.reshape(_NG * _C * _NP, d), w2,
                preferred_element_type=jnp.float32)
    M4 = M.reshape(_NG, _C, _NP, d) + combo[None]
    out = jax.nn.relu(jnp.max(M4, axis=1)) + bb
    pool = jnp.max(out[:, :_C], axis=1)
    return out.reshape(_NG * _NP, d), pool


def _main_kernel(h_ref, c1w_ref, g1_ref, bb1_ref, c2w_ref, g2_ref, bb2_ref,
                 p1_ref, p2_ref,
                 e1w1_ref, e1b1_ref, e1w2_ref, e1c_ref, e1b_ref,
                 e2w1_ref, e2b1_ref, e2w2_ref, e2c_ref, e2b_ref,
                 e3w1_ref, e3b1_ref, e3w2_ref, e3c_ref, e3b_ref,
                 l1w_ref, l1b_ref, l2w_ref, l2b_ref,
                 out_ref, acc_ref):
    zpad = jnp.zeros((_NP - _C, 128), jnp.float32)
    cols = []
    for q in range(_NG):
        h = h_ref[q]
        h = _conv30(h, c1w_ref[...], _TL)
        h = h * (g1_ref[...] * _BN_S) + bb1_ref[...]
        h = _elu(h)
        h = jnp.dot(h, p1_ref[...], preferred_element_type=jnp.float32)
        h = _conv30(h, c2w_ref[...], 100)
        h = h * (g2_ref[...] * _BN_S) + bb2_ref[...]
        h = _elu(h)
        h = jnp.dot(h, p2_ref[...], preferred_element_type=jnp.float32)
        cols.append(h)
        cols.append(zpad)
    nodes0 = jnp.concatenate(cols, axis=0)

    x1, pl1 = _edge_layer(nodes0, e1w1_ref[...], e1b1_ref[...],
                          e1w2_ref[...], e1c_ref[...], e1b_ref[...])
    x2, pl2 = _edge_layer(x1, e2w1_ref[...], e2b1_ref[...],
                          e2w2_ref[...], e2c_ref[...], e2b_ref[...])
    _, pl3 = _edge_layer(x2, e3w1_ref[...], e3b1_ref[...],
                         e3w2_ref[...], e3c_ref[...], e3b_ref[...])
    gid = pl.program_id(0)
    acc_ref[pl.ds(gid * _NG, _NG), :] = jnp.concatenate(
        [pl1, pl2, pl3], axis=1)

    @pl.when(gid == pl.num_programs(0) - 1)
    def _head():
        o = jnp.dot(acc_ref[...], l1w_ref[...],
                    preferred_element_type=jnp.float32)
        o = jax.nn.relu(o + l1b_ref[...])
        o = jnp.dot(o, l2w_ref[...], preferred_element_type=jnp.float32)
        out_ref[...] = jax.nn.relu(o + l2b_ref[...])


def _full(shape):
    nd = len(shape)
    return pl.BlockSpec(shape, lambda g, _n=nd: (0,) * _n)


def kernel(x, b1_conv_w, b1_bn_g, b1_bn_b, b2_conv_w, b2_bn_g, b2_bn_b,
           c1_w1, c1_b1, c1_w2, c1_b2, c1_bn_g, c1_bn_b,
           c2_w1, c2_b1, c2_w2, c2_b2, c2_bn_g, c2_bn_b,
           c3_w1, c3_b1, c3_w2, c3_b2, c3_bn_g, c3_bn_b,
           l1_w, l1_b, l2_w, l2_b):
    bsz = x.shape[0]
    ti, tj = np.triu_indices(_C, k=1)
    flat_idx = (np.arange(_FB)[:, None] * (_C * _C)
                + (ti * _C + tj)[None, :]).reshape(-1)
    h = x.reshape(bsz, _FB * _C * _C)[:, flat_idx].reshape(bsz, _C, _TL)

    c1w = jnp.transpose(b1_conv_w, (2, 0, 1))
    c2w = jnp.transpose(b2_conv_w, (2, 0, 1))
    g1 = b1_bn_g.reshape(_C, 1)
    bb1 = b1_bn_b.reshape(_C, 1)
    g2 = b2_bn_g.reshape(_C, 1)
    bb2 = b2_bn_b.reshape(_C, 1)

    row = lambda a: a.reshape(1, -1)
    s1 = c1_bn_g * _BN_S
    s2 = c2_bn_g * _BN_S
    s3 = c3_bn_g * _BN_S
    diag = np.where(np.arange(_C)[:, None] == np.arange(_NP)[None, :],
                    np.float32(_NEG), np.float32(0.0))
    combo = lambda b2s: b2s[None, None, :] + diag[:, :, None]
    operands = (
        h, c1w, g1, bb1, c2w, g2, bb2,
        jnp.asarray(_POOL1), jnp.asarray(_POOL2),
        c1_w1, row(c1_b1),
        (c1_w2 * s1[None, :]).astype(jnp.bfloat16), combo(c1_b2 * s1),
        row(c1_bn_b),
        c2_w1, row(c2_b1),
        (c2_w2 * s2[None, :]).astype(jnp.bfloat16), combo(c2_b2 * s2),
        row(c2_bn_b),
        c3_w1, row(c3_b1),
        (c3_w2 * s3[None, :]).astype(jnp.bfloat16), combo(c3_b2 * s3),
        row(c3_bn_b),
    )
    operands = operands + (l1_w, row(l1_b), l2_w, row(l2_b))
    in_specs = [pl.BlockSpec((_NG, _C, _TL), lambda g: (g, 0, 0))]
    in_specs += [_full(op.shape) for op in operands[1:]]

    ngrid = bsz // _NG
    out = pl.pallas_call(
        _main_kernel,
        grid=(ngrid,),
        in_specs=in_specs,
        out_specs=pl.BlockSpec((bsz, 128), lambda g: (0, 0)),
        out_shape=jax.ShapeDtypeStruct((bsz, 128), jnp.float32),
        scratch_shapes=[pltpu.VMEM((bsz, 896), jnp.float32)],
        compiler_params=pltpu.CompilerParams(
            dimension_semantics=("arbitrary",)),
    )(*operands)
    return out


# speedup vs baseline: 1.5559x; 1.0459x over previous
import math

import jax
import jax.numpy as jnp
import numpy as np
from jax.experimental import pallas as pl
from jax.experimental.pallas import tpu as pltpu

_B = 32
_C = 30
_FB = 4
_PLV = (_C * (_C - 1) // 2) * _FB
_TL = _PLV // _C
_NP = 32
_NG = 16
_BN_S = 1.0 / math.sqrt(1.0 + 1e-5)
_NEG = -1e30


def _pool_matrix(L, out_len):
    P = np.zeros((L, out_len), np.float32)
    for idx in range(out_len):
        s = (idx * L) // out_len
        e = ((idx + 1) * L + out_len - 1) // out_len
        P[s:e, idx] = 1.0 / (e - s)
    return P


_POOL1 = _pool_matrix(_TL, 100)
_POOL2 = _pool_matrix(100, 128)


def _elu(v):
    return jnp.where(v > 0, v, jnp.exp(v) - 1.0)


def _conv30(h, w, L):
    z = jnp.zeros((_C, 1), jnp.float32)
    hp = jnp.concatenate([z, h, z], axis=1)
    acc = jnp.dot(w[:, :, 0], hp[:, 0:L], preferred_element_type=jnp.float32)
    acc += jnp.dot(w[:, :, 1], hp[:, 1:L + 1],
                   preferred_element_type=jnp.float32)
    acc += jnp.dot(w[:, :, 2], hp[:, 2:L + 2],
                   preferred_element_type=jnp.float32)
    return acc


def _edge_layer(nodes, w1, b1, w2raw, b2, g, bb):
    d_in = nodes.shape[1]
    d = w2raw.shape[1]
    s = g * _BN_S
    w2 = (w2raw * s).astype(jnp.bfloat16)
    ii = jax.lax.broadcasted_iota(jnp.int32, (_C, _NP, 1), 0)
    jj = jax.lax.broadcasted_iota(jnp.int32, (_C, _NP, 1), 1)
    combo = jnp.where(ii == jj, _NEG, (b2 * s)[None])
    wt = w1[:d_in]
    wb = w1[d_in:]
    Bv = jnp.dot(nodes, wb, preferred_element_type=jnp.float32)
    A = jnp.dot(nodes, wt, preferred_element_type=jnp.float32) - Bv + b1
    Bv4 = Bv.reshape(_NG, _NP, 1, d)[:, :_C]
    A4 = A.reshape(_NG, 1, _NP, d)
    P = jax.nn.relu(Bv4 + A4).astype(jnp.bfloat16)
    M = jnp.dot(P.reshape(_NG * _C * _NP, d), w2,
                preferred_element_type=jnp.float32)
    M4 = M.reshape(_NG, _C, _NP, d) + combo[None]
    out = jax.nn.relu(jnp.max(M4, axis=1)) + bb
    pool = jnp.max(out[:, :_C], axis=1)
    return out.reshape(_NG * _NP, d), pool


def _main_kernel(h_ref, c1w_ref, g1_ref, bb1_ref, c2w_ref, g2_ref, bb2_ref,
                 p1_ref, p2_ref,
                 e1w1_ref, e1b1_ref, e1w2_ref, e1b2_ref, e1g_ref, e1b_ref,
                 e2w1_ref, e2b1_ref, e2w2_ref, e2b2_ref, e2g_ref, e2b_ref,
                 e3w1_ref, e3b1_ref, e3w2_ref, e3b2_ref, e3g_ref, e3b_ref,
                 l1w_ref, l1b_ref, l2w_ref, l2b_ref,
                 out_ref, acc_ref):
    zpad = jnp.zeros((_NP - _C, 128), jnp.float32)
    cols = []
    for q in range(_NG):
        h = h_ref[q]
        h = _conv30(h, c1w_ref[...], _TL)
        h = h * (g1_ref[...] * _BN_S) + bb1_ref[...]
        h = _elu(h)
        h = jnp.dot(h, p1_ref[...], preferred_element_type=jnp.float32)
        h = _conv30(h, c2w_ref[...], 100)
        h = h * (g2_ref[...] * _BN_S) + bb2_ref[...]
        h = _elu(h)
        h = jnp.dot(h, p2_ref[...], preferred_element_type=jnp.float32)
        cols.append(h)
        cols.append(zpad)
    nodes0 = jnp.concatenate(cols, axis=0)

    x1, pl1 = _edge_layer(nodes0, e1w1_ref[...], e1b1_ref[...],
                          e1w2_ref[...], e1b2_ref[...], e1g_ref[...],
                          e1b_ref[...])
    x2, pl2 = _edge_layer(x1, e2w1_ref[...], e2b1_ref[...],
                          e2w2_ref[...], e2b2_ref[...], e2g_ref[...],
                          e2b_ref[...])
    _, pl3 = _edge_layer(x2, e3w1_ref[...], e3b1_ref[...],
                         e3w2_ref[...], e3b2_ref[...], e3g_ref[...],
                         e3b_ref[...])
    gid = pl.program_id(0)
    acc_ref[pl.ds(gid * _NG, _NG), :] = jnp.concatenate(
        [pl1, pl2, pl3], axis=1)

    @pl.when(gid == pl.num_programs(0) - 1)
    def _head():
        o = jnp.dot(acc_ref[...], l1w_ref[...],
                    preferred_element_type=jnp.float32)
        o = jax.nn.relu(o + l1b_ref[...])
        o = jnp.dot(o, l2w_ref[...], preferred_element_type=jnp.float32)
        out_ref[...] = jax.nn.relu(o + l2b_ref[...])


def _full(shape):
    nd = len(shape)
    return pl.BlockSpec(shape, lambda g, _n=nd: (0,) * _n)


def kernel(x, b1_conv_w, b1_bn_g, b1_bn_b, b2_conv_w, b2_bn_g, b2_bn_b,
           c1_w1, c1_b1, c1_w2, c1_b2, c1_bn_g, c1_bn_b,
           c2_w1, c2_b1, c2_w2, c2_b2, c2_bn_g, c2_bn_b,
           c3_w1, c3_b1, c3_w2, c3_b2, c3_bn_g, c3_bn_b,
           l1_w, l1_b, l2_w, l2_b):
    bsz = x.shape[0]
    ti, tj = np.triu_indices(_C, k=1)
    flat_idx = (np.arange(_FB)[:, None] * (_C * _C)
                + (ti * _C + tj)[None, :]).reshape(-1)
    h = x.reshape(bsz, _FB * _C * _C)[:, flat_idx].reshape(bsz, _C, _TL)

    g1 = b1_bn_g.reshape(_C, 1)
    bb1 = b1_bn_b.reshape(_C, 1)
    g2 = b2_bn_g.reshape(_C, 1)
    bb2 = b2_bn_b.reshape(_C, 1)

    row = lambda a: a.reshape(1, -1)
    operands = (
        h, b1_conv_w, g1, bb1, b2_conv_w, g2, bb2,
        jnp.asarray(_POOL1), jnp.asarray(_POOL2),
        c1_w1, row(c1_b1), c1_w2, row(c1_b2), row(c1_bn_g), row(c1_bn_b),
        c2_w1, row(c2_b1), c2_w2, row(c2_b2), row(c2_bn_g), row(c2_bn_b),
        c3_w1, row(c3_b1), c3_w2, row(c3_b2), row(c3_bn_g), row(c3_bn_b),
    )
    operands = operands + (l1_w, row(l1_b), l2_w, row(l2_b))
    in_specs = [pl.BlockSpec((_NG, _C, _TL), lambda g: (g, 0, 0))]
    in_specs += [_full(op.shape) for op in operands[1:]]

    ngrid = bsz // _NG
    out = pl.pallas_call(
        _main_kernel,
        grid=(ngrid,),
        in_specs=in_specs,
        out_specs=pl.BlockSpec((bsz, 128), lambda g: (0, 0)),
        out_shape=jax.ShapeDtypeStruct((bsz, 128), jnp.float32),
        scratch_shapes=[pltpu.VMEM((bsz, 896), jnp.float32)],
        compiler_params=pltpu.CompilerParams(
            dimension_semantics=("arbitrary",)),
    )(*operands)
    return out
